# baseline (speedup 1.0000x reference)
import functools

import jax
import jax.numpy as jnp
from jax import lax
from jax.experimental import pallas as pl
from jax.experimental.pallas import tpu as pltpu
from jax.experimental.pallas import tpu_sc as plsc

_B = 4
_S = 8192
_D = 1024
_NC = 2
_NS = 16
_NW = _NC * _NS
_ROWS_PER_W = _S // _NW
_R = 4
_NCHUNK = _ROWS_PER_W // _R
_U = 8

_mesh = plsc.VectorSubcoreMesh(core_axis_name="c", subcore_axis_name="s")


@functools.partial(
    pl.kernel,
    mesh=_mesh,
    out_type=jax.ShapeDtypeStruct((_B, _S, _D), jnp.float32),
    scratch_types=[
        pltpu.VMEM((2, _R, _D), jnp.float32),
        pltpu.VMEM((2, _B, _R, _D), jnp.float32),
        pltpu.VMEM((2, _B, _R, _D), jnp.float32),
        pltpu.SemaphoreType.DMA,
        pltpu.SemaphoreType.DMA,
        pltpu.SemaphoreType.DMA,
        pltpu.SemaphoreType.DMA,
    ],
)
def _pos_add(x_hbm, w_hbm, out_hbm, wv, xv, ov, isem0, isem1, osem0, osem1):
    wid = lax.axis_index("s") * _NC + lax.axis_index("c")
    base = wid * _ROWS_PER_W
    isems = (isem0, isem1)
    osems = (osem0, osem1)

    def start_in(chunk, p):
        row = base + chunk * _R
        pltpu.async_copy(w_hbm.at[pl.ds(row, _R), :], wv.at[p], isems[p])
        for b in range(_B):
            pltpu.async_copy(
                x_hbm.at[b, pl.ds(row, _R), :], xv.at[p, b], isems[p]
            )

    def wait_in(p):
        pltpu.make_async_copy(
            w_hbm.at[pl.ds(0, _R), :], wv.at[p], isems[p]
        ).wait()
        for b in range(_B):
            pltpu.make_async_copy(
                x_hbm.at[b, pl.ds(0, _R), :], xv.at[p, b], isems[p]
            ).wait()

    def start_out(chunk, p):
        row = base + chunk * _R
        for b in range(_B):
            pltpu.async_copy(
                ov.at[p, b], out_hbm.at[b, pl.ds(row, _R), :], osems[p]
            )

    def wait_out(p):
        for b in range(_B):
            pltpu.make_async_copy(
                ov.at[p, b], out_hbm.at[b, pl.ds(0, _R), :], osems[p]
            ).wait()

    def compute(p):
        for r in range(_R):
            @plsc.parallel_loop(0, _D, step=16 * _U)
            def vec_body(s, _r=r):
                s = pl.multiple_of(s, 16 * _U)
                for u in range(_U):
                    su = s + u * 16
                    wvec = wv[p, _r, pl.ds(su, 16)]
                    for b in range(_B):
                        ov[p, b, _r, pl.ds(su, 16)] = (
                            xv[p, b, _r, pl.ds(su, 16)] + wvec
                        )

    def body(chunk, p, first_ring, last):
        wait_in(p)
        if not last:
            start_in(chunk + 1, p ^ 1)
        if not first_ring:
            wait_out(p)
        compute(p)
        start_out(chunk, p)

    start_in(0, 0)
    body(0, 0, True, False)
    body(1, 1, True, False)

    @pl.loop(2, _NCHUNK - 2, step=2)
    def _steady(g):
        body(g, 0, False, False)
        body(g + 1, 1, False, False)

    body(_NCHUNK - 2, 0, False, False)
    body(_NCHUNK - 1, 1, False, True)
    wait_out(0)
    wait_out(1)


def kernel(x, weight):
    return _pos_add(x, weight)


# speedup vs baseline: 1.4395x; 1.4395x over previous
import functools

import jax
import jax.numpy as jnp
from jax import lax
from jax.experimental import pallas as pl
from jax.experimental.pallas import tpu as pltpu
from jax.experimental.pallas import tpu_sc as plsc

_B = 4
_S = 8192
_D = 1024
_NC = 2
_NS = 16
_NW = _NC * _NS
_ROWS_PER_W = _S // _NW
_R = 8
_NCHUNK = _ROWS_PER_W // _R
_U = 8
_NBUF = 3

_mesh = plsc.VectorSubcoreMesh(core_axis_name="c", subcore_axis_name="s")


@functools.partial(
    pl.kernel,
    mesh=_mesh,
    out_type=jax.ShapeDtypeStruct((_B, _S, _D), jnp.float32),
    scratch_types=[
        pltpu.VMEM((_NBUF, _R, _D), jnp.float32),
        pltpu.VMEM((_NBUF, _B, _R, _D), jnp.float32),
        pltpu.SemaphoreType.DMA,
        pltpu.SemaphoreType.DMA,
        pltpu.SemaphoreType.DMA,
        pltpu.SemaphoreType.DMA,
        pltpu.SemaphoreType.DMA,
        pltpu.SemaphoreType.DMA,
    ],
)
def _pos_add(x_hbm, w_hbm, out_hbm, wv, xv, i0, i1, i2, o0, o1, o2):
    wid = lax.axis_index("s") * _NC + lax.axis_index("c")
    base = wid * _ROWS_PER_W
    isems = (i0, i1, i2)
    osems = (o0, o1, o2)

    def start_in(chunk, q):
        row = base + chunk * _R
        pltpu.async_copy(w_hbm.at[pl.ds(row, _R), :], wv.at[q], isems[q])
        for b in range(_B):
            pltpu.async_copy(
                x_hbm.at[b, pl.ds(row, _R), :], xv.at[q, b], isems[q]
            )

    def wait_in(q):
        pltpu.make_async_copy(
            w_hbm.at[pl.ds(0, _R), :], wv.at[q], isems[q]
        ).wait()
        for b in range(_B):
            pltpu.make_async_copy(
                x_hbm.at[b, pl.ds(0, _R), :], xv.at[q, b], isems[q]
            ).wait()

    def start_out(chunk, q):
        row = base + chunk * _R
        for b in range(_B):
            pltpu.async_copy(
                xv.at[q, b], out_hbm.at[b, pl.ds(row, _R), :], osems[q]
            )

    def wait_out(q):
        for b in range(_B):
            pltpu.make_async_copy(
                xv.at[q, b], out_hbm.at[b, pl.ds(0, _R), :], osems[q]
            ).wait()

    def compute(q):
        def row_body(r, c):
            @plsc.parallel_loop(0, _D, step=16 * _U)
            def vec_body(s):
                s = pl.multiple_of(s, 16 * _U)
                for u in range(_U):
                    su = s + u * 16
                    wvec = wv[q, r, pl.ds(su, 16)]
                    for b in range(_B):
                        xv[q, b, r, pl.ds(su, 16)] = (
                            xv[q, b, r, pl.ds(su, 16)] + wvec
                        )
            return c

        lax.fori_loop(0, _R, row_body, 0)

    def body(chunk, q, first_ring, last):
        wait_in(q)
        if not first_ring:
            wait_out((q + 1) % _NBUF)
        if not last:
            start_in(chunk + 1, (q + 1) % _NBUF)
        compute(q)
        start_out(chunk, q)

    start_in(0, 0)
    body(0, 0, True, False)
    body(1, 1, True, False)

    @pl.loop(2, _NCHUNK - 3, step=3)
    def _steady(g):
        body(g, 2, False, False)
        body(g + 1, 0, False, False)
        body(g + 2, 1, False, False)

    body(_NCHUNK - 3, 2, False, False)
    body(_NCHUNK - 2, 0, False, False)
    body(_NCHUNK - 1, 1, False, True)
    wait_out(0)
    wait_out(1)


def kernel(x, weight):
    return _pos_add(x, weight)


# speedup vs baseline: 1.9620x; 1.3629x over previous
import jax
import jax.numpy as jnp
from jax.experimental import pallas as pl
from jax.experimental.pallas import tpu as pltpu

_B = 4
_S = 8192
_D = 1024
_SBLK = 256


def _body(x_ref, w_ref, o_ref):
    o_ref[...] = x_ref[...] + w_ref[...][None, :, :]


@jax.jit
def _pos_add(x, w):
    return pl.pallas_call(
        _body,
        grid=(_S // _SBLK,),
        in_specs=[
            pl.BlockSpec((_B, _SBLK, _D), lambda i: (0, i, 0)),
            pl.BlockSpec((_SBLK, _D), lambda i: (i, 0)),
        ],
        out_specs=pl.BlockSpec((_B, _SBLK, _D), lambda i: (0, i, 0)),
        out_shape=jax.ShapeDtypeStruct((_B, _S, _D), jnp.float32),
        compiler_params=pltpu.CompilerParams(
            dimension_semantics=("arbitrary",),
        ),
    )(x, w)


def kernel(x, weight):
    return _pos_add(x, weight)
